# baseline (speedup 1.0000x reference)
import jax
import jax.numpy as jnp
from jax import lax
from jax.experimental import pallas as pl
from jax.experimental.pallas import tpu as pltpu
from jax.experimental.pallas import tpu_sc as plsc

USER_N = 5000
ITEM_N = 5000
N = USER_N + ITEM_N
D = 128
DH = 64
NLAYERS = 3
E = 320000
NC = 2
NS = 16
BLK = 512
QC = 128
NQ = BLK // QC
NBLK = 40
EPT = NBLK * BLK
E_PAD = NS * EPT
NP = 10240
RPT = NP // NS
RC = 128
ZR = 32
NEB = 3


def _sc_body(emb_hbm, src_hbm, dst_hbm, vals_hbm, out_hbm, cur_hbm,
             srcb, dstb, vbuf, gbuf, abuf, zbuf, b_sh, esem, gsem, ssem):
    c_id = lax.axis_index("c")
    s_id = lax.axis_index("s")
    base = s_id * RPT
    cbase = c_id * NP + base

    def issue_e(k, slot):
        pltpu.async_copy(src_hbm.at[c_id, s_id, k], srcb.at[slot],
                         esem.at[slot])
        pltpu.async_copy(dst_hbm.at[s_id, k], dstb.at[slot], esem.at[slot])
        pltpu.async_copy(vals_hbm.at[s_id, k], vbuf.at[slot], esem.at[slot])

    def wait_e(k, slot):
        pltpu.make_async_copy(src_hbm.at[c_id, s_id, k], srcb.at[slot],
                              esem.at[slot]).wait()
        pltpu.make_async_copy(dst_hbm.at[s_id, k], dstb.at[slot],
                              esem.at[slot]).wait()
        pltpu.make_async_copy(vals_hbm.at[s_id, k], vbuf.at[slot],
                              esem.at[slot]).wait()

    def issue_g(slot, bb):
        pltpu.async_copy(cur_hbm.at[srcb.at[slot]], gbuf.at[bb], gsem.at[bb])

    def wait_g(bb):
        pltpu.make_async_copy(cur_hbm.at[srcb.at[0]], gbuf.at[bb],
                              gsem.at[bb]).wait()

    def issue_s(slot, q, bb):
        pltpu.async_copy(gbuf.at[bb, pl.ds(q * QC, QC)],
                         b_sh.at[dstb.at[slot, q]], ssem.at[q], add=True)

    def wait_s(q):
        pltpu.make_async_copy(gbuf.at[0, pl.ds(0, QC)],
                              b_sh.at[dstb.at[0, 0]], ssem.at[q]).wait()

    pltpu.sync_copy(emb_hbm.at[pl.ds(cbase, RPT)], out_hbm.at[pl.ds(cbase, RPT)])
    pltpu.sync_copy(emb_hbm.at[pl.ds(cbase, RPT)], cur_hbm.at[pl.ds(cbase, RPT)])

    def _zb(i, carry):
        for q in range(4):
            zbuf[i, pl.ds(q * 16, 16)] = jnp.zeros((16,), jnp.float32)
        return carry
    lax.fori_loop(0, ZR, _zb, 0)
    for k in range(RPT // ZR):
        pltpu.sync_copy(zbuf, b_sh.at[pl.ds(base + k * ZR, ZR)])
    plsc.subcore_barrier()

    def _quarter(bb, slot, q):
        def _sixteen(t, carry3):
            vv = vbuf[slot, pl.ds(q * QC + t * 16, 16)]
            for i in range(16):
                v = jnp.take_along_axis(
                    vv, jnp.full((16,), i, jnp.int32), axis=0)
                r = q * QC + t * 16 + i
                for h in range(4):
                    sl = pl.ds(h * 16, 16)
                    gbuf[bb, r, sl] = gbuf[bb, r, sl] * v
            return carry3
        lax.fori_loop(0, QC // 16, _sixteen, 0)

    def _layer(l, carry):
        issue_e(0, 0)
        issue_e(1, 1)
        wait_e(0, 0)
        issue_g(0, 0)

        def _block(k, carry2):
            bb = lax.rem(k, 2)
            slot = lax.rem(k, NEB)
            slot1 = lax.rem(k + 1, NEB)
            @pl.when(k + 2 < NBLK)
            def _():
                issue_e(k + 2, lax.rem(k + 2, NEB))
            wait_g(bb)
            for q in range(2):
                @pl.when(k >= 1)
                def _():
                    wait_s(q)
                _quarter(bb, slot, q)
                issue_s(slot, q, bb)
            @pl.when(k >= 1)
            def _():
                wait_s(2)
                wait_s(3)

            @pl.when(k + 1 < NBLK)
            def _():
                wait_e(k + 1, slot1)
                issue_g(slot1, 1 - bb)
            for q in range(2, 4):
                _quarter(bb, slot, q)
                issue_s(slot, q, bb)
            return carry2
        lax.fori_loop(0, NBLK, _block, 0)
        for q in range(4):
            wait_s(q)
        plsc.subcore_barrier()

        for k in range(RPT // RC):
            rb = base + k * RC
            cb = cbase + k * RC
            pltpu.sync_copy(b_sh.at[pl.ds(rb, RC)], gbuf.at[0, pl.ds(0, RC)])
            for z in range(RC // ZR):
                pltpu.sync_copy(zbuf, b_sh.at[pl.ds(rb + z * ZR, ZR)])
            pltpu.sync_copy(out_hbm.at[pl.ds(cb, RC)], abuf)

            def _acc(i, carry3):
                for h in range(4):
                    sl = pl.ds(h * 16, 16)
                    abuf[i, sl] = abuf[i, sl] + gbuf[0, i, sl]
                return carry3
            lax.fori_loop(0, RC, _acc, 0)
            pltpu.sync_copy(abuf, out_hbm.at[pl.ds(cb, RC)])
            pltpu.sync_copy(gbuf.at[0, pl.ds(0, RC)],
                            cur_hbm.at[pl.ds(cb, RC)])
        plsc.subcore_barrier()
        return carry

    lax.fori_loop(0, NLAYERS, _layer, 0)


def _make_call():
    mesh = plsc.VectorSubcoreMesh(core_axis_name="c", subcore_axis_name="s",
                                  num_cores=NC, num_subcores=NS)
    return pl.kernel(
        _sc_body,
        out_type=(
            jax.ShapeDtypeStruct((NC * NP, DH), jnp.float32),
            jax.ShapeDtypeStruct((NC * NP, DH), jnp.float32),
        ),
        mesh=mesh,
        compiler_params=pltpu.CompilerParams(use_tc_tiling_on_sc=False),
        scratch_types=[
            pltpu.VMEM((NEB, BLK), jnp.int32),
            pltpu.VMEM((NEB, NQ, QC), jnp.int32),
            pltpu.VMEM((NEB, BLK), jnp.float32),
            pltpu.VMEM((2, BLK, DH), jnp.float32),
            pltpu.VMEM((RC, DH), jnp.float32),
            pltpu.VMEM((ZR, DH), jnp.float32),
            pltpu.VMEM_SHARED((NP, DH), jnp.float32),
            pltpu.SemaphoreType.DMA((NEB,)),
            pltpu.SemaphoreType.DMA((2,)),
            pltpu.SemaphoreType.DMA((NQ,)),
        ],
    )


_sc_call = _make_call()


def kernel(adj_edge_index, adj_edge_values, uEmbeds, iEmbeds):
    embeds = jnp.concatenate([uEmbeds, iEmbeds], axis=0)
    rpad = jnp.zeros((NP - N, DH), jnp.float32)
    emb_flat = jnp.concatenate(
        [embeds[:, :DH], rpad, embeds[:, DH:], rpad], axis=0)

    dst = adj_edge_index[0]
    src = adj_edge_index[1]
    npad = E_PAD - E
    pad_idx = (jnp.arange(npad, dtype=jnp.int32) * 61) % N
    src_p = jnp.concatenate([src, pad_idx])
    dst_p = jnp.concatenate([dst, pad_idx])
    vals_p = jnp.concatenate([adj_edge_values,
                              jnp.zeros((npad,), jnp.float32)])

    src_a = jnp.stack([src_p, src_p + NP]).reshape(NC, NS, NBLK, BLK)
    dst_a = dst_p.reshape(NS, NBLK, NQ, QC)
    vals_a = vals_p.reshape(NS, NBLK, BLK)

    out_flat, _ = _sc_call(emb_flat, src_a, dst_a, vals_a)
    out = jnp.concatenate([out_flat[:N], out_flat[NP:NP + N]], axis=1)
    return (out[:USER_N], out[USER_N:])


# speedup vs baseline: 1.1869x; 1.1869x over previous
import jax
import jax.numpy as jnp
from jax import lax
from jax.experimental import pallas as pl
from jax.experimental.pallas import tpu as pltpu
from jax.experimental.pallas import tpu_sc as plsc

USER_N = 5000
ITEM_N = 5000
N = USER_N + ITEM_N
D = 128
DH = 64
NLAYERS = 3
E = 320000
NC = 2
NS = 16
BLK = 256
QC = 128
NQ = BLK // QC
NBLK = 80
EPT = NBLK * BLK
E_PAD = NS * EPT
NP = 10240
RPT = NP // NS
RC = 128
ZR = 32
NEB = 3


def _sc_body(emb_hbm, src_hbm, dst_hbm, vals_hbm, out_hbm,
             srcb, dstb, vbuf, gbuf, abuf, zbuf, b_sh, cur_sh,
             esem, gsem, ssem):
    c_id = lax.axis_index("c")
    s_id = lax.axis_index("s")
    base = s_id * RPT
    cbase = c_id * NP + base

    def issue_e(k, slot):
        pltpu.async_copy(src_hbm.at[s_id, k], srcb.at[slot], esem.at[slot])
        pltpu.async_copy(dst_hbm.at[s_id, k], dstb.at[slot], esem.at[slot])
        pltpu.async_copy(vals_hbm.at[s_id, k], vbuf.at[slot], esem.at[slot])

    def wait_e(k, slot):
        pltpu.make_async_copy(src_hbm.at[s_id, k], srcb.at[slot],
                              esem.at[slot]).wait()
        pltpu.make_async_copy(dst_hbm.at[s_id, k], dstb.at[slot],
                              esem.at[slot]).wait()
        pltpu.make_async_copy(vals_hbm.at[s_id, k], vbuf.at[slot],
                              esem.at[slot]).wait()

    def issue_g(slot, bb):
        pltpu.async_copy(cur_sh.at[srcb.at[slot]], gbuf.at[bb], gsem.at[bb])

    def wait_g(bb):
        pltpu.make_async_copy(cur_sh.at[srcb.at[0]], gbuf.at[bb],
                              gsem.at[bb]).wait()

    def issue_s(slot, q, bb):
        pltpu.async_copy(gbuf.at[bb, pl.ds(q * QC, QC)],
                         b_sh.at[dstb.at[slot, q]], ssem.at[q], add=True)

    def wait_s(q):
        pltpu.make_async_copy(gbuf.at[0, pl.ds(0, QC)],
                              b_sh.at[dstb.at[0, 0]], ssem.at[q]).wait()

    pltpu.sync_copy(emb_hbm.at[pl.ds(cbase, RPT)], out_hbm.at[pl.ds(cbase, RPT)])
    pltpu.sync_copy(emb_hbm.at[pl.ds(cbase, RPT)], cur_sh.at[pl.ds(base, RPT)])

    def _zb(i, carry):
        for q in range(4):
            zbuf[i, pl.ds(q * 16, 16)] = jnp.zeros((16,), jnp.float32)
        return carry
    lax.fori_loop(0, ZR, _zb, 0)
    for k in range(RPT // ZR):
        pltpu.sync_copy(zbuf, b_sh.at[pl.ds(base + k * ZR, ZR)])
    plsc.subcore_barrier()

    def _quarter(bb, slot, q):
        def _sixteen(t, carry3):
            vv = vbuf[slot, pl.ds(q * QC + t * 16, 16)]
            for i in range(16):
                v = jnp.take_along_axis(
                    vv, jnp.full((16,), i, jnp.int32), axis=0)
                r = q * QC + t * 16 + i
                for h in range(4):
                    sl = pl.ds(h * 16, 16)
                    gbuf[bb, r, sl] = gbuf[bb, r, sl] * v
            return carry3
        lax.fori_loop(0, QC // 16, _sixteen, 0)

    def _layer(l, carry):
        issue_e(0, 0)
        issue_e(1, 1)
        wait_e(0, 0)
        issue_g(0, 0)

        def _block(k, carry2):
            bb = lax.rem(k, 2)
            slot = lax.rem(k, NEB)
            slot1 = lax.rem(k + 1, NEB)
            @pl.when(k + 2 < NBLK)
            def _():
                issue_e(k + 2, lax.rem(k + 2, NEB))
            wait_g(bb)
            @pl.when(k >= 1)
            def _():
                wait_s(0)
            _quarter(bb, slot, 0)
            issue_s(slot, 0, bb)
            @pl.when(k >= 1)
            def _():
                wait_s(1)

            @pl.when(k + 1 < NBLK)
            def _():
                wait_e(k + 1, slot1)
                issue_g(slot1, 1 - bb)
            _quarter(bb, slot, 1)
            issue_s(slot, 1, bb)
            return carry2
        lax.fori_loop(0, NBLK, _block, 0)
        for q in range(NQ):
            wait_s(q)
        plsc.subcore_barrier()

        for k in range(RPT // RC):
            rb = base + k * RC
            cb = cbase + k * RC
            pltpu.sync_copy(b_sh.at[pl.ds(rb, RC)], gbuf.at[0, pl.ds(0, RC)])
            for z in range(RC // ZR):
                pltpu.sync_copy(zbuf, b_sh.at[pl.ds(rb + z * ZR, ZR)])
            pltpu.sync_copy(out_hbm.at[pl.ds(cb, RC)], abuf)

            def _acc(i, carry3):
                for h in range(4):
                    sl = pl.ds(h * 16, 16)
                    abuf[i, sl] = abuf[i, sl] + gbuf[0, i, sl]
                return carry3
            lax.fori_loop(0, RC, _acc, 0)
            pltpu.sync_copy(abuf, out_hbm.at[pl.ds(cb, RC)])
            pltpu.sync_copy(gbuf.at[0, pl.ds(0, RC)],
                            cur_sh.at[pl.ds(rb, RC)])
        plsc.subcore_barrier()
        return carry

    lax.fori_loop(0, NLAYERS, _layer, 0)


def _make_call():
    mesh = plsc.VectorSubcoreMesh(core_axis_name="c", subcore_axis_name="s",
                                  num_cores=NC, num_subcores=NS)
    return pl.kernel(
        _sc_body,
        out_type=jax.ShapeDtypeStruct((NC * NP, DH), jnp.float32),
        mesh=mesh,
        compiler_params=pltpu.CompilerParams(use_tc_tiling_on_sc=False),
        scratch_types=[
            pltpu.VMEM((NEB, BLK), jnp.int32),
            pltpu.VMEM((NEB, NQ, QC), jnp.int32),
            pltpu.VMEM((NEB, BLK), jnp.float32),
            pltpu.VMEM((2, BLK, DH), jnp.float32),
            pltpu.VMEM((RC, DH), jnp.float32),
            pltpu.VMEM((ZR, DH), jnp.float32),
            pltpu.VMEM_SHARED((NP, DH), jnp.float32),
            pltpu.VMEM_SHARED((NP, DH), jnp.float32),
            pltpu.SemaphoreType.DMA((NEB,)),
            pltpu.SemaphoreType.DMA((2,)),
            pltpu.SemaphoreType.DMA((NQ,)),
        ],
    )


_sc_call = _make_call()


def kernel(adj_edge_index, adj_edge_values, uEmbeds, iEmbeds):
    embeds = jnp.concatenate([uEmbeds, iEmbeds], axis=0)
    rpad = jnp.zeros((NP - N, DH), jnp.float32)
    emb_flat = jnp.concatenate(
        [embeds[:, :DH], rpad, embeds[:, DH:], rpad], axis=0)

    dst = adj_edge_index[0]
    src = adj_edge_index[1]
    npad = E_PAD - E
    pad_idx = (jnp.arange(npad, dtype=jnp.int32) * 61) % N
    src_p = jnp.concatenate([src, pad_idx])
    dst_p = jnp.concatenate([dst, pad_idx])
    vals_p = jnp.concatenate([adj_edge_values,
                              jnp.zeros((npad,), jnp.float32)])

    src_a = src_p.reshape(NS, NBLK, BLK)
    dst_a = dst_p.reshape(NS, NBLK, NQ, QC)
    vals_a = vals_p.reshape(NS, NBLK, BLK)

    out_flat = _sc_call(emb_flat, src_a, dst_a, vals_a)
    out = jnp.concatenate([out_flat[:N], out_flat[NP:NP + N]], axis=1)
    return (out[:USER_N], out[USER_N:])
